# baseline (speedup 1.0000x reference)
import jax
import jax.numpy as jnp
from jax.experimental import pallas as pl
from jax.experimental.pallas import tpu as pltpu

PADDING_IDX = 0

_BLOCK_B = 64


def _matmul_kernel(x_ref, w_ref, o_ref):
    w = w_ref[...]
    row_ids = jax.lax.broadcasted_iota(jnp.int32, w.shape, 0)
    w = jnp.where(row_ids == PADDING_IDX, 0.0, w).astype(jnp.bfloat16)
    for j in range(x_ref.shape[0]):
        x = x_ref[j].astype(jnp.bfloat16)
        o_ref[j] = jnp.dot(x, w, preferred_element_type=jnp.float32)


def kernel(input, weight):
    b, l, v = input.shape
    d = weight.shape[1]
    grid = (b // _BLOCK_B,)
    return pl.pallas_call(
        _matmul_kernel,
        grid=grid,
        in_specs=[
            pl.BlockSpec((_BLOCK_B, l, v), lambda i: (i, 0, 0)),
            pl.BlockSpec((v, d), lambda i: (0, 0)),
        ],
        out_specs=pl.BlockSpec((_BLOCK_B, l, d), lambda i: (i, 0, 0)),
        out_shape=jax.ShapeDtypeStruct((b, l, d), jnp.float32),
        compiler_params=pltpu.CompilerParams(
            dimension_semantics=("parallel",),
        ),
    )(input, weight)
